# baseline (speedup 1.0000x reference)
import jax
import jax.numpy as jnp
from jax.experimental import pallas as pl

NFEAT = 256
NHID = 256
TURNS = 6
ALPHA = 0.2
TOPK = 3
B_GRAPHS = 4


def _ident_body(x_ref, o_ref):
    o_ref[...] = x_ref[...]


def _pallas_ident(x):
    return pl.pallas_call(
        _ident_body,
        out_shape=jax.ShapeDtypeStruct(x.shape, x.dtype),
    )(x)


def kernel(emb, nmask, nodeidx, edge_index, tid, uid, gid,
           W, a_src, a_dst, Wz, Wr, Wc, Uz, Ur, Uc, bz, br, bc):
    B, T, Nt, F = emb.shape
    flat = (emb * nmask[..., None]).reshape(B * T * Nt, F)
    feat = flat[nodeidx]
    N = feat.shape[0]
    Wh = feat @ W
    hp = jnp.zeros((N, Wh.shape[1]), dtype=feat.dtype)
    src = edge_index[0]
    dst = edge_index[1]
    attn = jnp.zeros((src.shape[0],), dtype=feat.dtype)
    for t in range(TURNS):
        s = Wh + hp
        e = jax.nn.leaky_relu(s[src] @ a_src + s[dst] @ a_dst, negative_slope=ALPHA)
        m = jax.ops.segment_max(e, dst, num_segments=N)
        ee = jnp.exp(e - m[dst])
        den = jax.ops.segment_sum(ee, dst, num_segments=N)
        attn = ee / (den[dst] + 1e-9)
        msg = jax.ops.segment_sum(attn[:, None] * s[src], dst, num_segments=N)
        z = jax.nn.sigmoid(msg @ Wz + hp @ Uz + bz)
        r = jax.nn.sigmoid(msg @ Wr + hp @ Ur + br)
        hcand = jnp.tanh(msg @ Wc + (r * hp) @ Uc + bc)
        hnew = (1.0 - z) * hp + z * hcand
        upd = (tid == t)[:, None]
        hp = jnp.where(upd, hnew, hp)
    attn_score = jax.ops.segment_sum(attn, dst, num_segments=N)

    def _top(b, u):
        mask = (gid == b) & (uid == u)
        sc = jnp.where(mask, attn_score, -jnp.inf)
        idx = jax.lax.top_k(sc, TOPK)[1]
        return hp[idx].reshape(1, -1)

    x1 = jnp.stack([_top(b, 0) for b in range(B_GRAPHS)], axis=0)
    x2 = jnp.stack([_top(b, 1) for b in range(B_GRAPHS)], axis=0)
    return (_pallas_ident(x1), _pallas_ident(x2))


# speedup vs baseline: 1.1945x; 1.1945x over previous
import functools

import jax
import jax.numpy as jnp
import numpy as np
from jax import lax
from jax.experimental import pallas as pl
from jax.experimental.pallas import tpu as pltpu
from jax.experimental.pallas import tpu_sc as plsc

N = 10000
E = 160000
F = 256
TURNS = 6
ALPHA = 0.2
TOPK = 3
B_GRAPHS = 4

NW = 32
EPAD = E + 512
NPADT = 10016

_HALF = [5040 * k for k in range(1, 12)] + [55440 + 4928 * k for k in range(1, 5)]
_BOUNDS = _HALF + [80000] + [80000 + b for b in _HALF]
_BFLAG = np.zeros(EPAD, np.int32)
_BFLAG[np.array(_BOUNDS, np.int64)] = 1

_mesh = plsc.VectorSubcoreMesh(core_axis_name="c", subcore_axis_name="s")


def _wid():
    return lax.axis_index("s") * 2 + lax.axis_index("c")


def _flat_body(e_ref, m_ref, o_ref):
    o_ref[...] = e_ref[...] * m_ref[...]


def _flat_mask(emb2, nm2):
    return pl.pallas_call(
        _flat_body,
        grid=(6,),
        in_specs=[pl.BlockSpec((2048, F), lambda i: (i, 0)),
                  pl.BlockSpec((2048, 1), lambda i: (i, 0))],
        out_specs=pl.BlockSpec((2048, F), lambda i: (i, 0)),
        out_shape=jax.ShapeDtypeStruct((12288, F), jnp.float32),
    )(emb2, nm2)


def _mm_body(a_ref, b_ref, o_ref):
    o_ref[...] = jnp.dot(a_ref[...], b_ref[...], preferred_element_type=jnp.float32)


def _matmul(a, b):
    m, k = a.shape
    n = b.shape[1]
    return pl.pallas_call(
        _mm_body,
        grid=(5,),
        in_specs=[pl.BlockSpec((m // 5, k), lambda i: (i, 0)),
                  pl.BlockSpec((k, n), lambda i: (0, 0))],
        out_specs=pl.BlockSpec((m // 5, n), lambda i: (i, 0)),
        out_shape=jax.ShapeDtypeStruct((m, n), jnp.float32),
    )(a, b)


def _dense1_body(wh_ref, hp_ref, s_ref):
    s_ref[...] = wh_ref[...] + hp_ref[...]


def _dense1(Wh, hp):
    return pl.pallas_call(
        _dense1_body,
        grid=(5,),
        in_specs=[pl.BlockSpec((2000, F), lambda i: (i, 0)),
                  pl.BlockSpec((2000, F), lambda i: (i, 0))],
        out_specs=pl.BlockSpec((2000, F), lambda i: (i, 0)),
        out_shape=jax.ShapeDtypeStruct((N, F), jnp.float32),
    )(Wh, hp)


def _gru_body(t, msg_ref, hp_ref, tid_ref, wz_ref, wc_ref, bz_ref, bc_ref, o_ref):
    m1 = jnp.dot(msg_ref[...], wz_ref[...], preferred_element_type=jnp.float32)
    z = jax.nn.sigmoid(m1 + bz_ref[...])
    m2 = jnp.dot(msg_ref[...], wc_ref[...], preferred_element_type=jnp.float32)
    hc = jnp.tanh(m2 + bc_ref[...])
    hp = hp_ref[...]
    hnew = (1.0 - z) * hp + z * hc
    o_ref[...] = jnp.where(tid_ref[...] == t, hnew, hp)


def _gru(t, msg, hp, tid2, Wz, Wc, bz2, bc2):
    return pl.pallas_call(
        functools.partial(_gru_body, t),
        grid=(5,),
        in_specs=[pl.BlockSpec((2000, F), lambda i: (i, 0)),
                  pl.BlockSpec((2000, F), lambda i: (i, 0)),
                  pl.BlockSpec((2000, 1), lambda i: (i, 0)),
                  pl.BlockSpec((F, F), lambda i: (0, 0)),
                  pl.BlockSpec((F, F), lambda i: (0, 0)),
                  pl.BlockSpec((1, F), lambda i: (0, 0)),
                  pl.BlockSpec((1, F), lambda i: (0, 0))],
        out_specs=pl.BlockSpec((2000, F), lambda i: (i, 0)),
        out_shape=jax.ShapeDtypeStruct((N, F), jnp.float32),
    )(msg, hp, tid2, Wz, Wc, bz2, bc2)


def _topk_body(sc_ref, gid_ref, uid_ref, o_ref):
    iota = lax.broadcasted_iota(jnp.int32, (1, 10240), 1)
    lane = lax.broadcasted_iota(jnp.int32, (1, 128), 1)
    score = sc_ref[...]
    gid = gid_ref[...]
    uid = uid_ref[...]
    for b in range(B_GRAPHS):
        for u in range(2):
            g = b * 2 + u
            mask = (gid == b) & (uid == u)
            v = jnp.where(mask, score, jnp.float32(-1.0))
            row = jnp.zeros((1, 128), jnp.int32)
            for k in range(TOPK):
                mx = jnp.max(v)
                sel = jnp.where(v == mx, iota, jnp.int32(2 ** 30))
                idx = jnp.min(sel)
                v = jnp.where(iota == idx, jnp.float32(-3.0), v)
                row = jnp.where(lane == k, idx, row)
            o_ref[g : g + 1, :] = row


def _topk(score_p, gid_p, uid_p):
    return pl.pallas_call(
        _topk_body,
        out_shape=jax.ShapeDtypeStruct((8, 128), jnp.int32),
    )(score_p, gid_p, uid_p)


@functools.partial(
    pl.kernel, mesh=_mesh,
    compiler_params=pltpu.CompilerParams(needs_layout_passes=False),
    out_type=jax.ShapeDtypeStruct((10240, F), jnp.float32),
    scratch_types=[
        pltpu.VMEM((320,), jnp.int32),
        pltpu.VMEM((64, F), jnp.float32),
        pltpu.SemaphoreType.DMA,
    ],
)
def _sc_feat(flat_hbm, idx_hbm, out_hbm, idx_v, rows_v, sem):
    w = _wid()
    base = w * 320
    pltpu.sync_copy(idx_hbm.at[pl.ds(base, 320)], idx_v)
    def blk(k, _):
        pltpu.async_copy(flat_hbm.at[idx_v.at[pl.ds(k * 64, 64)]], rows_v, sem).wait()
        pltpu.sync_copy(rows_v, out_hbm.at[pl.ds(base + k * 64, 64)])
        return 0
    lax.fori_loop(0, 5, blk, 0)


@functools.partial(
    pl.kernel, mesh=_mesh,
    compiler_params=pltpu.CompilerParams(needs_layout_passes=False),
    out_type=jax.ShapeDtypeStruct((EPAD,), jnp.float32),
    scratch_types=[
        pltpu.VMEM((NPADT,), jnp.float32),
        pltpu.VMEM((512,), jnp.float32),
        pltpu.VMEM((512,), jnp.int32),
        pltpu.VMEM((512,), jnp.float32),
    ],
)
def _sc_edge_ee(m_hbm, e_hbm, sdst_hbm, ee_hbm, m_v, e_v, dst_v, ee_v):
    w = _wid()
    pltpu.sync_copy(m_hbm, m_v)
    ebase = w * 4992
    ecnt = jnp.where(w == 31, 5248, 4992)
    nblk = (ecnt + 511) // 512
    def blk(k, _):
        b0 = ebase + k * 512
        pltpu.sync_copy(e_hbm.at[pl.ds(b0, 512)], e_v)
        pltpu.sync_copy(sdst_hbm.at[pl.ds(b0, 512)], dst_v)
        def ch(i, _):
            sl = pl.ds(i * 16, 16)
            mg = plsc.load_gather(m_v, [dst_v[sl]])
            ee_v[sl] = jnp.exp(e_v[sl] - mg)
            return 0
        lax.fori_loop(0, 32, ch, 0)
        pltpu.sync_copy(ee_v, ee_hbm.at[pl.ds(b0, 512)])
        return 0
    lax.fori_loop(0, nblk, blk, 0)


def _make_sc_msg(t):
    attn_mode = (t == TURNS - 1)

    @functools.partial(
        pl.kernel, mesh=_mesh,
        compiler_params=pltpu.CompilerParams(needs_layout_passes=False),
        out_type=[jax.ShapeDtypeStruct((N, F), jnp.float32),
                  jax.ShapeDtypeStruct((EPAD,), jnp.float32)],
        scratch_types=[
            pltpu.VMEM((NPADT,), jnp.int32),
            pltpu.VMEM((NPADT,), jnp.int32),
            pltpu.VMEM((NPADT,), jnp.float32),
            pltpu.VMEM((16,), jnp.int32),
            pltpu.VMEM((16,), jnp.int32),
            pltpu.VMEM((16,), jnp.float32),
            pltpu.VMEM((16,), jnp.int32),
            pltpu.VMEM((16,), jnp.float32),
            pltpu.VMEM((16, F), jnp.float32),
            pltpu.VMEM((F,), jnp.float32),
            pltpu.VMEM((F,), jnp.float32),
            pltpu.SemaphoreType.DMA,
        ],
    )
    def _sc_msg(starts_hbm, tid_hbm, den_hbm, ssrc_hbm, sdst_hbm, ee_hbm,
                bf_hbm, s_hbm, msg_hbm, attn_hbm,
                st_v, tid_v, den_v, src_v, dst_v, ee_v, bf_v, at_v,
                rows_v, acc_v, tot_v, sem):
        w = _wid()
        pltpu.sync_copy(starts_hbm, st_v)
        pltpu.sync_copy(tid_hbm, tid_v)
        pltpu.sync_copy(den_hbm, den_v)
        nlo = w * 312
        ncnt = jnp.where(w == 31, 328, 312)

        def node(d, _):
            sv = st_v[pl.ds(d, 16)]
            r0 = sv[0]
            r1 = sv[1]
            td = tid_v[pl.ds(d, 16)][0]
            masked = td == t
            for i in range(F // 16):
                sl = pl.ds(i * 16, 16)
                acc_v[sl] = jnp.zeros((16,), jnp.float32)
                tot_v[sl] = jnp.zeros((16,), jnp.float32)
            c0f = (r0 // 8) * 8
            nch = (r1 - c0f + 15) // 16
            if not attn_mode:
                nch = jnp.where(masked, nch, 0)

            def chunk(ci, _):
                c0 = c0f + ci * 16
                pltpu.sync_copy(sdst_hbm.at[pl.ds(c0, 16)], dst_v)
                pltpu.sync_copy(ee_hbm.at[pl.ds(c0, 16)], ee_v)
                dg = plsc.load_gather(den_v, [dst_v[...]])
                attn = ee_v[...] / (dg + np.float32(1e-9))
                if attn_mode:
                    at_v[...] = attn
                    pltpu.sync_copy(at_v, attn_hbm.at[pl.ds(c0, 16)])

                @pl.when(masked)
                def _():
                    pltpu.sync_copy(ssrc_hbm.at[pl.ds(c0, 16)], src_v)
                    pltpu.sync_copy(bf_hbm.at[pl.ds(c0, 16)], bf_v)
                    lanepos = c0 + lax.iota(jnp.int32, 16)
                    valid = (lanepos >= r0) & (lanepos < r1)
                    gidx = jnp.where(valid, src_v[...], 0)
                    pltpu.async_copy(s_hbm.at[gidx], rows_v, sem).wait()
                    bfv = bf_v[...]
                    for j in range(16):
                        p = c0 + j
                        inrun = (p >= r0) & (p < r1)

                        @pl.when(inrun & (bfv[j] == 1) & (p > r0))
                        def _(j=j):
                            for i in range(F // 16):
                                sl = pl.ds(i * 16, 16)
                                tot_v[sl] = tot_v[sl] + acc_v[sl]
                                acc_v[sl] = jnp.zeros((16,), jnp.float32)

                        @pl.when(inrun)
                        def _(j=j):
                            aj = jnp.full((16,), attn[j], jnp.float32)
                            for i in range(F // 16):
                                sl = pl.ds(i * 16, 16)
                                acc_v[sl] = acc_v[sl] + aj * rows_v[j, sl]
                return 0

            lax.fori_loop(0, nch, chunk, 0)

            @pl.when(masked)
            def _():
                for i in range(F // 16):
                    sl = pl.ds(i * 16, 16)
                    tot_v[sl] = tot_v[sl] + acc_v[sl]
                pltpu.sync_copy(tot_v, msg_hbm.at[d])
            return 0

        lax.fori_loop(nlo, nlo + ncnt, node, 0)

    return _sc_msg


_SC_MSG = [_make_sc_msg(t) for t in range(TURNS)]


def kernel(emb, nmask, nodeidx, edge_index, tid, uid, gid,
           W, a_src, a_dst, Wz, Wr, Wc, Uz, Ur, Uc, bz, br, bc):
    emb2 = emb.reshape(12288, F)
    nm2 = nmask.reshape(12288, 1)
    src = edge_index[0]
    dst = edge_index[1]
    perm = jnp.argsort(dst, stable=True)
    inv_perm = jnp.argsort(perm, stable=True)
    sdst = dst[perm]
    ssrc = src[perm]
    starts = jnp.searchsorted(sdst, jnp.arange(N + 1), side="left").astype(jnp.int32)
    starts_p = jnp.pad(starts, (0, NPADT - N - 1))
    tid_p = jnp.pad(tid, (0, NPADT - N))
    ssrc_p = jnp.pad(ssrc, (0, EPAD - E))
    sdst_p = jnp.pad(sdst, (0, EPAD - E))
    bflag = jnp.asarray(_BFLAG)
    idx_p = jnp.pad(nodeidx, (0, 10240 - N), mode="edge")

    flat = _flat_mask(emb2, nm2)
    feat = _sc_feat(flat, idx_p)[:N]
    Wh = _matmul(feat, W)

    hp = jnp.zeros((N, F), jnp.float32)
    bz2 = bz.reshape(1, F)
    bc2 = bc.reshape(1, F)
    tid2 = tid.reshape(N, 1)
    attn_sorted = None

    for t in range(TURNS):
        s = _dense1(Wh, hp)
        e = jax.nn.leaky_relu(s[src] @ a_src + s[dst] @ a_dst,
                              negative_slope=ALPHA)
        m = jax.ops.segment_max(e, dst, num_segments=N)
        m_p = jnp.pad(m, (0, NPADT - N))
        e_sorted = jnp.pad(e[perm], (0, EPAD - E))
        ee_sorted = _sc_edge_ee(m_p, e_sorted, sdst_p)
        den = jax.ops.segment_sum(ee_sorted[:E][inv_perm], dst, num_segments=N)
        attn_orig = (ee_sorted[:E] / (den[sdst] + 1e-9))[inv_perm]
        msg = jax.ops.segment_sum(attn_orig[:, None] * s[src], dst,
                                  num_segments=N)
        if t == TURNS - 1:
            attn_last = attn_orig
        hp = _gru(t, msg, hp, tid2, Wz, Wc, bz2, bc2)

    score = jax.ops.segment_sum(attn_last, dst, num_segments=N)
    score_p = jnp.pad(score, (0, 10240 - N), constant_values=-2.0).reshape(1, 10240)
    gid_p = jnp.pad(gid, (0, 10240 - N), constant_values=-1).reshape(1, 10240)
    uid_p = jnp.pad(uid, (0, 10240 - N), constant_values=-1).reshape(1, 10240)
    idx8 = _topk(score_p, gid_p, uid_p)[:, :TOPK]

    rows = hp[idx8.reshape(-1)]
    rows = rows.reshape(B_GRAPHS, 2, 1, TOPK * F)
    x1 = rows[:, 0]
    x2 = rows[:, 1]
    return (x1, x2)


# speedup vs baseline: 1.1945x; 1.0000x over previous
import functools

import jax
import jax.numpy as jnp
import numpy as np
from jax import lax
from jax.experimental import pallas as pl
from jax.experimental.pallas import tpu as pltpu
from jax.experimental.pallas import tpu_sc as plsc

N = 10000
E = 160000
F = 256
TURNS = 6
ALPHA = 0.2
TOPK = 3
B_GRAPHS = 4

NW = 32
EPAD = E + 512
NPADT = 10016

_HALF = [5040 * k for k in range(1, 12)] + [55440 + 4928 * k for k in range(1, 5)]
_BOUNDS = _HALF + [80000] + [80000 + b for b in _HALF]
_BFLAG = np.zeros(EPAD, np.int32)
_BFLAG[np.array(_BOUNDS, np.int64)] = 1

_mesh = plsc.VectorSubcoreMesh(core_axis_name="c", subcore_axis_name="s")


def _wid():
    return lax.axis_index("s") * 2 + lax.axis_index("c")


def _flat_body(e_ref, m_ref, o_ref):
    o_ref[...] = e_ref[...] * m_ref[...]


def _flat_mask(emb2, nm2):
    return pl.pallas_call(
        _flat_body,
        grid=(6,),
        in_specs=[pl.BlockSpec((2048, F), lambda i: (i, 0)),
                  pl.BlockSpec((2048, 1), lambda i: (i, 0))],
        out_specs=pl.BlockSpec((2048, F), lambda i: (i, 0)),
        out_shape=jax.ShapeDtypeStruct((12288, F), jnp.float32),
    )(emb2, nm2)


def _mm_body(a_ref, b_ref, o_ref):
    o_ref[...] = jnp.dot(a_ref[...], b_ref[...], preferred_element_type=jnp.float32)


def _matmul(a, b):
    m, k = a.shape
    n = b.shape[1]
    return pl.pallas_call(
        _mm_body,
        grid=(5,),
        in_specs=[pl.BlockSpec((m // 5, k), lambda i: (i, 0)),
                  pl.BlockSpec((k, n), lambda i: (0, 0))],
        out_specs=pl.BlockSpec((m // 5, n), lambda i: (i, 0)),
        out_shape=jax.ShapeDtypeStruct((m, n), jnp.float32),
    )(a, b)


def _dense1_body(wh_ref, hp_ref, s_ref):
    s_ref[...] = wh_ref[...] + hp_ref[...]


def _dense1(Wh, hp):
    return pl.pallas_call(
        _dense1_body,
        grid=(5,),
        in_specs=[pl.BlockSpec((2000, F), lambda i: (i, 0)),
                  pl.BlockSpec((2000, F), lambda i: (i, 0))],
        out_specs=pl.BlockSpec((2000, F), lambda i: (i, 0)),
        out_shape=jax.ShapeDtypeStruct((N, F), jnp.float32),
    )(Wh, hp)


def _gru_body(t, msg_ref, hp_ref, tid_ref, wz_ref, wc_ref, bz_ref, bc_ref, o_ref):
    m1 = jnp.dot(msg_ref[...], wz_ref[...], preferred_element_type=jnp.float32)
    z = jax.nn.sigmoid(m1 + bz_ref[...])
    m2 = jnp.dot(msg_ref[...], wc_ref[...], preferred_element_type=jnp.float32)
    hc = jnp.tanh(m2 + bc_ref[...])
    hp = hp_ref[...]
    hnew = (1.0 - z) * hp + z * hc
    o_ref[...] = jnp.where(tid_ref[...] == t, hnew, hp)


def _gru(t, msg, hp, tid2, Wz, Wc, bz2, bc2):
    return pl.pallas_call(
        functools.partial(_gru_body, t),
        grid=(5,),
        in_specs=[pl.BlockSpec((2000, F), lambda i: (i, 0)),
                  pl.BlockSpec((2000, F), lambda i: (i, 0)),
                  pl.BlockSpec((2000, 1), lambda i: (i, 0)),
                  pl.BlockSpec((F, F), lambda i: (0, 0)),
                  pl.BlockSpec((F, F), lambda i: (0, 0)),
                  pl.BlockSpec((1, F), lambda i: (0, 0)),
                  pl.BlockSpec((1, F), lambda i: (0, 0))],
        out_specs=pl.BlockSpec((2000, F), lambda i: (i, 0)),
        out_shape=jax.ShapeDtypeStruct((N, F), jnp.float32),
    )(msg, hp, tid2, Wz, Wc, bz2, bc2)


def _topk_body(sc_ref, gid_ref, uid_ref, o_ref):
    iota = lax.broadcasted_iota(jnp.int32, (1, 10240), 1)
    lane = lax.broadcasted_iota(jnp.int32, (1, 128), 1)
    score = sc_ref[...]
    gid = gid_ref[...]
    uid = uid_ref[...]
    for b in range(B_GRAPHS):
        for u in range(2):
            g = b * 2 + u
            mask = (gid == b) & (uid == u)
            v = jnp.where(mask, score, jnp.float32(-1.0))
            row = jnp.zeros((1, 128), jnp.int32)
            for k in range(TOPK):
                mx = jnp.max(v)
                sel = jnp.where(v == mx, iota, jnp.int32(2 ** 30))
                idx = jnp.min(sel)
                v = jnp.where(iota == idx, jnp.float32(-3.0), v)
                row = jnp.where(lane == k, idx, row)
            o_ref[g : g + 1, :] = row


def _topk(score_p, gid_p, uid_p):
    return pl.pallas_call(
        _topk_body,
        out_shape=jax.ShapeDtypeStruct((8, 128), jnp.int32),
    )(score_p, gid_p, uid_p)


@functools.partial(
    pl.kernel, mesh=_mesh,
    compiler_params=pltpu.CompilerParams(needs_layout_passes=False),
    out_type=jax.ShapeDtypeStruct((10240, F), jnp.float32),
    scratch_types=[
        pltpu.VMEM((320,), jnp.int32),
        pltpu.VMEM((64, F), jnp.float32),
        pltpu.SemaphoreType.DMA,
    ],
)
def _sc_feat(flat_hbm, idx_hbm, out_hbm, idx_v, rows_v, sem):
    w = _wid()
    base = w * 320
    pltpu.sync_copy(idx_hbm.at[pl.ds(base, 320)], idx_v)
    def blk(k, _):
        pltpu.async_copy(flat_hbm.at[idx_v.at[pl.ds(k * 64, 64)]], rows_v, sem).wait()
        pltpu.sync_copy(rows_v, out_hbm.at[pl.ds(base + k * 64, 64)])
        return 0
    lax.fori_loop(0, 5, blk, 0)


@functools.partial(
    pl.kernel, mesh=_mesh,
    compiler_params=pltpu.CompilerParams(needs_layout_passes=False),
    out_type=jax.ShapeDtypeStruct((EPAD,), jnp.float32),
    scratch_types=[
        pltpu.VMEM((NPADT,), jnp.float32),
        pltpu.VMEM((512,), jnp.float32),
        pltpu.VMEM((512,), jnp.int32),
        pltpu.VMEM((512,), jnp.float32),
    ],
)
def _sc_edge_ee(m_hbm, e_hbm, sdst_hbm, ee_hbm, m_v, e_v, dst_v, ee_v):
    w = _wid()
    pltpu.sync_copy(m_hbm, m_v)
    ebase = w * 4992
    ecnt = jnp.where(w == 31, 5248, 4992)
    nblk = (ecnt + 511) // 512
    def blk(k, _):
        b0 = ebase + k * 512
        pltpu.sync_copy(e_hbm.at[pl.ds(b0, 512)], e_v)
        pltpu.sync_copy(sdst_hbm.at[pl.ds(b0, 512)], dst_v)
        def ch(i, _):
            sl = pl.ds(i * 16, 16)
            mg = plsc.load_gather(m_v, [dst_v[sl]])
            ee_v[sl] = jnp.exp(e_v[sl] - mg)
            return 0
        lax.fori_loop(0, 32, ch, 0)
        pltpu.sync_copy(ee_v, ee_hbm.at[pl.ds(b0, 512)])
        return 0
    lax.fori_loop(0, nblk, blk, 0)


def kernel(emb, nmask, nodeidx, edge_index, tid, uid, gid,
           W, a_src, a_dst, Wz, Wr, Wc, Uz, Ur, Uc, bz, br, bc):
    emb2 = emb.reshape(12288, F)
    nm2 = nmask.reshape(12288, 1)
    src = edge_index[0]
    dst = edge_index[1]
    perm = jnp.argsort(dst, stable=True)
    inv_perm = jnp.argsort(perm, stable=True)
    sdst = dst[perm]
    ssrc = src[perm]
    starts = jnp.searchsorted(sdst, jnp.arange(N + 1), side="left").astype(jnp.int32)
    starts_p = jnp.pad(starts, (0, NPADT - N - 1))
    tid_p = jnp.pad(tid, (0, NPADT - N))
    ssrc_p = jnp.pad(ssrc, (0, EPAD - E))
    sdst_p = jnp.pad(sdst, (0, EPAD - E))
    bflag = jnp.asarray(_BFLAG)
    idx_p = jnp.pad(nodeidx, (0, 10240 - N), mode="edge")

    flat = _flat_mask(emb2, nm2)
    feat = _sc_feat(flat, idx_p)[:N]
    Wh = _matmul(feat, W)

    hp = jnp.zeros((N, F), jnp.float32)
    bz2 = bz.reshape(1, F)
    bc2 = bc.reshape(1, F)
    tid2 = tid.reshape(N, 1)
    attn_sorted = None

    for t in range(TURNS):
        s = _dense1(Wh, hp)
        e = jax.nn.leaky_relu(s[src] @ a_src + s[dst] @ a_dst,
                              negative_slope=ALPHA)
        m = jax.ops.segment_max(e, dst, num_segments=N)
        m_p = jnp.pad(m, (0, NPADT - N))
        e_sorted = jnp.pad(e[perm], (0, EPAD - E))
        ee_sorted = _sc_edge_ee(m_p, e_sorted, sdst_p)
        den = jax.ops.segment_sum(ee_sorted[:E][inv_perm], dst, num_segments=N)
        attn_orig = (ee_sorted[:E] / (den[sdst] + 1e-9))[inv_perm]
        msg = jax.ops.segment_sum(attn_orig[:, None] * s[src], dst,
                                  num_segments=N)
        if t == TURNS - 1:
            attn_last = attn_orig
        hp = _gru(t, msg, hp, tid2, Wz, Wc, bz2, bc2)

    score = jax.ops.segment_sum(attn_last, dst, num_segments=N)
    score_p = jnp.pad(score, (0, 10240 - N), constant_values=-2.0).reshape(1, 10240)
    gid_p = jnp.pad(gid, (0, 10240 - N), constant_values=-1).reshape(1, 10240)
    uid_p = jnp.pad(uid, (0, 10240 - N), constant_values=-1).reshape(1, 10240)
    idx8 = _topk(score_p, gid_p, uid_p)[:, :TOPK]

    rows = hp[idx8.reshape(-1)]
    rows = rows.reshape(B_GRAPHS, 2, 1, TOPK * F)
    x1 = rows[:, 0]
    x2 = rows[:, 1]
    return (x1, x2)
